# baseline (speedup 1.0000x reference)
import dataclasses
import functools

import numpy as np

import jax
import jax.numpy as jnp
from jax import lax
from jax.experimental import pallas as pl
from jax.experimental.pallas import tpu as pltpu
from jax.experimental.pallas import tpu_sc as plsc

_VOCAB = 262144
_DIM = 128
_SEQ = 2048
_B = 32
_EPS = 1e-5

_NC = 2
_NS = 16
_NW = _NC * _NS
_L = 16

_CHUNK = 128

_PACK_SRC = np.empty((_DIM,), dtype=np.int32)
for _g in range(_DIM // 32):
    _base = 32 * _g
    _PACK_SRC[_base + 0:_base + 32:2] = np.arange(16) + _base
    _PACK_SRC[_base + 1:_base + 32:2] = np.arange(16) + _base + 16


def _sc_gather_bf16(tok, idx2d, split, nsplit):
    n_rows = idx2d.shape[0] * _CHUNK // nsplit
    n_chunks = n_rows // _CHUNK
    chunks_per_w = n_chunks // _NW
    split_base = split * n_chunks
    depth = min(2, chunks_per_w)
    mesh = plsc.VectorSubcoreMesh(core_axis_name="c", subcore_axis_name="s")
    cp = pltpu.CompilerParams()
    if "needs_layout_passes" in pltpu.CompilerParams.__dataclass_fields__:
        cp = dataclasses.replace(cp, needs_layout_passes=False)

    @functools.partial(
        pl.kernel,
        mesh=mesh,
        compiler_params=cp,
        out_type=pltpu.HBM((n_rows, _DIM // 2), jnp.int32),
        scratch_types=[
            pltpu.VMEM((chunks_per_w, _CHUNK), jnp.int32),
            pltpu.VMEM((depth, _CHUNK, _DIM), jnp.float32),
            pltpu.VMEM((depth, _CHUNK, _DIM // 2), jnp.int32),
            pltpu.SemaphoreType.DMA((depth,)),
            pltpu.SemaphoreType.DMA((depth,)),
        ],
    )
    def k(table_hbm, idx_hbm, out_hbm, idx_v, rows_v, pk_v, sem_g, sem_w):
        wid = lax.axis_index("s") * _NC + lax.axis_index("c")
        cbase = wid * chunks_per_w
        pltpu.sync_copy(
            idx_hbm.at[pl.ds(split_base + cbase, chunks_per_w)], idx_v
        )

        def start_gather(j):
            b = j % depth
            return pltpu.async_copy(
                table_hbm.at[idx_v.at[j]], rows_v.at[b], sem_g.at[b]
            )

        def start_write(j):
            b = j % depth
            return pltpu.async_copy(
                pk_v.at[b],
                out_hbm.at[pl.ds((cbase + j) * _CHUNK, _CHUNK)],
                sem_w.at[b],
            )

        def pack_chunk(b):
            @pl.loop(0, _CHUNK)
            def _(r):
                for g in range(_DIM // 32):
                    a = rows_v[b, r, pl.ds(32 * g, _L)]
                    bb = rows_v[b, r, pl.ds(32 * g + _L, _L)]
                    p = plsc.pack(a, bb, format=plsc.PackFormat.INTERLEAVED)
                    pk_v[b, r, pl.ds(_L * g, _L)] = plsc.bitcast(p, jnp.int32)

        gathers = {j: start_gather(j) for j in range(depth)}
        writes = {}
        for j in range(chunks_per_w):
            gathers[j].wait()
            pack_chunk(j % depth)
            writes[j] = start_write(j)
            if j + depth < chunks_per_w:
                writes[j].wait()
                gathers[j + depth] = start_gather(j + depth)
        for j in range(max(0, chunks_per_w - depth), chunks_per_w):
            writes[j].wait()

    return k(tok, idx2d)


def _ln_body_full(prev_ref, emb_ref, pos_ref, w_ref, b_ref, cm_ref, jm_ref, o_ref):
    del prev_ref
    _ln_compute(emb_ref, pos_ref, w_ref, b_ref, cm_ref, jm_ref, o_ref)


def _ln_body(emb_ref, pos_ref, w_ref, b_ref, cm_ref, jm_ref, o_ref):
    _ln_compute(emb_ref, pos_ref, w_ref, b_ref, cm_ref, jm_ref, o_ref)


def _ln_compute(emb_ref, pos_ref, w_ref, b_ref, cm_ref, jm_ref, o_ref):
    e = emb_ref[...] + pos_ref[...]
    dot = lambda a, b: lax.dot_general(
        a, b, (((1,), (0,)), ((), ())), preferred_element_type=jnp.float32
    )
    c = dot(e, cm_ref[...])
    v = dot((c * c).astype(jnp.bfloat16), jm_ref[...])
    o_ref[...] = c * lax.rsqrt(v + _EPS) * w_ref[...] + b_ref[...]


def _tc_pos_ln_chunk(gathered_c, pos_p, ln_w, ln_b, prev, block_off, n_total):
    grid = gathered_c.shape[0] // _SEQ
    jm = jnp.full((_DIM, _DIM), 1.0 / _DIM, dtype=jnp.bfloat16)
    cm = (
        jnp.eye(_DIM, dtype=jnp.float32)[_PACK_SRC] - 1.0 / _DIM
    ).astype(jnp.bfloat16)
    in_specs = [
        pl.BlockSpec((_SEQ, _DIM), lambda i: (i, 0)),
        pl.BlockSpec((_SEQ, _DIM), lambda i: (0, 0)),
        pl.BlockSpec((1, _DIM), lambda i: (0, 0)),
        pl.BlockSpec((1, _DIM), lambda i: (0, 0)),
        pl.BlockSpec((_DIM, _DIM), lambda i: (0, 0)),
        pl.BlockSpec((_DIM, _DIM), lambda i: (0, 0)),
    ]
    args = (gathered_c, pos_p, ln_w.reshape(1, _DIM), ln_b.reshape(1, _DIM),
            cm, jm)
    body = _ln_body
    aliases = {}
    if prev is not None:
        in_specs = [pl.BlockSpec(memory_space=pltpu.MemorySpace.HBM)] + in_specs
        args = (prev,) + args
        body = _ln_body_full
        aliases = {0: 0}
    return pl.pallas_call(
        body,
        grid=(grid,),
        in_specs=in_specs,
        out_specs=pl.BlockSpec((_SEQ, _DIM), lambda i: (i + block_off, 0)),
        out_shape=jax.ShapeDtypeStruct((n_total, _DIM), jnp.float32),
        input_output_aliases=aliases,
        compiler_params=pltpu.CompilerParams(
            dimension_semantics=("arbitrary",)
        ),
    )(*args)


_NSPLIT = 4


def kernel(x, tok, pos, ln_w, ln_b):
    b, seq = x.shape
    n_rows = b * seq
    rows_per_split = n_rows // _NSPLIT
    idx2d = x.reshape(n_rows // _CHUNK, _CHUNK)
    gathered = [
        _sc_gather_bf16(tok, idx2d, c, _NSPLIT) for c in range(_NSPLIT)
    ]
    gathered = [
        lax.bitcast_convert_type(g, jnp.bfloat16).reshape(-1, _DIM)
        for g in gathered
    ]
    pos_p = pos[:, _PACK_SRC].astype(jnp.bfloat16)
    blocks_per_split = rows_per_split // _SEQ
    out = None
    for c in range(_NSPLIT):
        out = _tc_pos_ln_chunk(
            gathered[c], pos_p, ln_w, ln_b, out, c * blocks_per_split, n_rows
        )
    return out.reshape(b, seq, _DIM)


# speedup vs baseline: 2.2288x; 2.2288x over previous
import functools

import jax
import jax.numpy as jnp
from jax import lax
from jax.experimental import pallas as pl
from jax.experimental.pallas import tpu as pltpu
from jax.experimental.pallas import tpu_sc as plsc

_VOCAB = 262144
_DIM = 128
_SEQ = 2048
_B = 32
_EPS = 1e-5

_NC = 2
_NS = 16
_NW = _NC * _NS

_CHUNK = 128


def _sc_gather(tok, idx2d, split, nsplit):
    n_rows = idx2d.shape[0] * _CHUNK // nsplit
    n_chunks = n_rows // _CHUNK
    chunks_per_w = n_chunks // _NW
    split_base = split * n_chunks
    depth = min(6, chunks_per_w)
    mesh = plsc.VectorSubcoreMesh(core_axis_name="c", subcore_axis_name="s")

    @functools.partial(
        pl.kernel,
        mesh=mesh,
        out_type=jax.ShapeDtypeStruct((n_rows, _DIM), jnp.float32),
        scratch_types=[
            pltpu.VMEM((chunks_per_w, _CHUNK), jnp.int32),
            pltpu.VMEM((depth, _CHUNK, _DIM), jnp.float32),
            pltpu.SemaphoreType.DMA((depth,)),
            pltpu.SemaphoreType.DMA((depth,)),
        ],
    )
    def k(table_hbm, idx_hbm, out_hbm, idx_v, rows_v, sem_g, sem_w):
        wid = lax.axis_index("s") * _NC + lax.axis_index("c")
        cbase = wid * chunks_per_w
        pltpu.sync_copy(
            idx_hbm.at[pl.ds(split_base + cbase, chunks_per_w)], idx_v
        )

        def start_gather(j):
            b = j % depth
            return pltpu.async_copy(
                table_hbm.at[idx_v.at[j]], rows_v.at[b], sem_g.at[b]
            )

        def start_write(j):
            b = j % depth
            return pltpu.async_copy(
                rows_v.at[b],
                out_hbm.at[pl.ds((cbase + j) * _CHUNK, _CHUNK)],
                sem_w.at[b],
            )

        gathers = {j: start_gather(j) for j in range(depth)}
        writes = {}
        for j in range(chunks_per_w):
            gathers[j].wait()
            writes[j] = start_write(j)
            if j + depth < chunks_per_w:
                writes[j].wait()
                gathers[j + depth] = start_gather(j + depth)
        for j in range(max(0, chunks_per_w - depth), chunks_per_w):
            writes[j].wait()

    return k(tok, idx2d)


def _ln_body_full(prev_ref, emb_ref, pos_ref, w_ref, b_ref, cm_ref, jm_ref, o_ref):
    del prev_ref
    _ln_compute(emb_ref, pos_ref, w_ref, b_ref, cm_ref, jm_ref, o_ref)


def _ln_body(emb_ref, pos_ref, w_ref, b_ref, cm_ref, jm_ref, o_ref):
    _ln_compute(emb_ref, pos_ref, w_ref, b_ref, cm_ref, jm_ref, o_ref)


def _ln_compute(emb_ref, pos_ref, w_ref, b_ref, cm_ref, jm_ref, o_ref):
    e = emb_ref[...] + pos_ref[...]
    dot = lambda a, b: lax.dot_general(
        a.astype(jnp.bfloat16),
        b,
        (((1,), (0,)), ((), ())),
        preferred_element_type=jnp.float32,
    )
    c = dot(e, cm_ref[...])
    v = dot(c * c, jm_ref[...])
    o_ref[...] = c * lax.rsqrt(v + _EPS) * w_ref[...] + b_ref[...]


def _tc_pos_ln_chunk(gathered_c, pos, ln_w, ln_b, prev, block_off, n_total):
    blk = 1024
    grid = gathered_c.shape[0] // blk
    jm = jnp.full((_DIM, _DIM), 1.0 / _DIM, dtype=jnp.bfloat16)
    cm = (jnp.eye(_DIM, dtype=jnp.float32) - 1.0 / _DIM).astype(jnp.bfloat16)
    in_specs = [
        pl.BlockSpec((blk, _DIM), lambda i: (i, 0)),
        pl.BlockSpec((blk, _DIM), lambda i: (i % (_SEQ // blk), 0)),
        pl.BlockSpec((1, _DIM), lambda i: (0, 0)),
        pl.BlockSpec((1, _DIM), lambda i: (0, 0)),
        pl.BlockSpec((_DIM, _DIM), lambda i: (0, 0)),
        pl.BlockSpec((_DIM, _DIM), lambda i: (0, 0)),
    ]
    args = (gathered_c, pos, ln_w.reshape(1, _DIM), ln_b.reshape(1, _DIM),
            cm, jm)
    body = _ln_body
    aliases = {}
    if prev is not None:
        in_specs = [pl.BlockSpec(memory_space=pltpu.MemorySpace.HBM)] + in_specs
        args = (prev,) + args
        body = _ln_body_full
        aliases = {0: 0}
    return pl.pallas_call(
        body,
        grid=(grid,),
        in_specs=in_specs,
        out_specs=pl.BlockSpec((blk, _DIM), lambda i: (i + block_off, 0)),
        out_shape=jax.ShapeDtypeStruct((n_total, _DIM), jnp.float32),
        input_output_aliases=aliases,
        compiler_params=pltpu.CompilerParams(
            dimension_semantics=("arbitrary",)
        ),
    )(*args)


_NSPLIT = 4


def kernel(x, tok, pos, ln_w, ln_b):
    b, seq = x.shape
    n_rows = b * seq
    rows_per_split = n_rows // _NSPLIT
    idx2d = x.reshape(n_rows // _CHUNK, _CHUNK)
    gathered = [
        _sc_gather(tok, idx2d, c, _NSPLIT) for c in range(_NSPLIT)
    ]
    blocks_per_split = rows_per_split // 1024
    out = None
    for c in range(_NSPLIT):
        out = _tc_pos_ln_chunk(
            gathered[c], pos, ln_w, ln_b, out, c * blocks_per_split, n_rows
        )
    return out.reshape(b, seq, _DIM)


# speedup vs baseline: 2.9439x; 1.3208x over previous
import functools

import jax
import jax.numpy as jnp
from jax import lax
from jax.experimental import pallas as pl
from jax.experimental.pallas import tpu as pltpu
from jax.experimental.pallas import tpu_sc as plsc

_VOCAB = 262144
_DIM = 128
_SEQ = 2048
_B = 32
_EPS = 1e-5

_NC = 2
_NS = 16
_NW = _NC * _NS

_CHUNK = 128


def _sc_gather(tok, idx2d, split, nsplit):
    n_rows = idx2d.shape[0] * _CHUNK // nsplit
    n_chunks = n_rows // _CHUNK
    chunks_per_w = n_chunks // _NW
    split_base = split * n_chunks
    depth = min(6, chunks_per_w)
    mesh = plsc.VectorSubcoreMesh(core_axis_name="c", subcore_axis_name="s")

    @functools.partial(
        pl.kernel,
        mesh=mesh,
        out_type=jax.ShapeDtypeStruct((n_rows, _DIM), jnp.float32),
        scratch_types=[
            pltpu.VMEM((chunks_per_w, _CHUNK), jnp.int32),
            pltpu.VMEM((depth, _CHUNK, _DIM), jnp.float32),
            pltpu.SemaphoreType.DMA((depth,)),
            pltpu.SemaphoreType.DMA((depth,)),
        ],
    )
    def k(table_hbm, idx_hbm, out_hbm, idx_v, rows_v, sem_g, sem_w):
        wid = lax.axis_index("s") * _NC + lax.axis_index("c")
        cbase = wid * chunks_per_w
        pltpu.sync_copy(
            idx_hbm.at[pl.ds(split_base + cbase, chunks_per_w)], idx_v
        )

        def start_gather(j):
            b = j % depth
            return pltpu.async_copy(
                table_hbm.at[idx_v.at[j]], rows_v.at[b], sem_g.at[b]
            )

        def start_write(j):
            b = j % depth
            return pltpu.async_copy(
                rows_v.at[b],
                out_hbm.at[pl.ds((cbase + j) * _CHUNK, _CHUNK)],
                sem_w.at[b],
            )

        gathers = {j: start_gather(j) for j in range(depth)}
        writes = {}
        for j in range(chunks_per_w):
            gathers[j].wait()
            writes[j] = start_write(j)
            if j + depth < chunks_per_w:
                writes[j].wait()
                gathers[j + depth] = start_gather(j + depth)
        for j in range(max(0, chunks_per_w - depth), chunks_per_w):
            writes[j].wait()

    return k(tok, idx2d)


def _ln_body_full(prev_ref, emb_ref, pos_ref, w_ref, b_ref, cm_ref, jm_ref, o_ref):
    del prev_ref
    _ln_compute(emb_ref, pos_ref, w_ref, b_ref, cm_ref, jm_ref, o_ref)


def _ln_body(emb_ref, pos_ref, w_ref, b_ref, cm_ref, jm_ref, o_ref):
    _ln_compute(emb_ref, pos_ref, w_ref, b_ref, cm_ref, jm_ref, o_ref)


def _ln_compute(emb_ref, pos_ref, w_ref, b_ref, cm_ref, jm_ref, o_ref):
    e = emb_ref[...] + pos_ref[...]
    dot = lambda a, b: lax.dot_general(
        a.astype(jnp.bfloat16),
        b,
        (((1,), (0,)), ((), ())),
        preferred_element_type=jnp.float32,
    )
    c = dot(e, cm_ref[...])
    v = dot(c * c, jm_ref[...])
    o_ref[...] = c * lax.rsqrt(v + _EPS) * w_ref[...] + b_ref[...]


def _tc_pos_ln_chunk(gathered_c, pos, ln_w, ln_b, prev, block_off, n_total):
    grid = gathered_c.shape[0] // _SEQ
    jm = jnp.full((_DIM, _DIM), 1.0 / _DIM, dtype=jnp.bfloat16)
    cm = (jnp.eye(_DIM, dtype=jnp.float32) - 1.0 / _DIM).astype(jnp.bfloat16)
    in_specs = [
        pl.BlockSpec((_SEQ, _DIM), lambda i: (i, 0)),
        pl.BlockSpec((_SEQ, _DIM), lambda i: (0, 0)),
        pl.BlockSpec((1, _DIM), lambda i: (0, 0)),
        pl.BlockSpec((1, _DIM), lambda i: (0, 0)),
        pl.BlockSpec((_DIM, _DIM), lambda i: (0, 0)),
        pl.BlockSpec((_DIM, _DIM), lambda i: (0, 0)),
    ]
    args = (gathered_c, pos, ln_w.reshape(1, _DIM), ln_b.reshape(1, _DIM),
            cm, jm)
    body = _ln_body
    aliases = {}
    if prev is not None:
        in_specs = [pl.BlockSpec(memory_space=pltpu.MemorySpace.HBM)] + in_specs
        args = (prev,) + args
        body = _ln_body_full
        aliases = {0: 0}
    return pl.pallas_call(
        body,
        grid=(grid,),
        in_specs=in_specs,
        out_specs=pl.BlockSpec((_SEQ, _DIM), lambda i: (i + block_off, 0)),
        out_shape=jax.ShapeDtypeStruct((n_total, _DIM), jnp.float32),
        input_output_aliases=aliases,
        compiler_params=pltpu.CompilerParams(
            dimension_semantics=("arbitrary",)
        ),
    )(*args)


_NSPLIT = 4


def kernel(x, tok, pos, ln_w, ln_b):
    b, seq = x.shape
    n_rows = b * seq
    rows_per_split = n_rows // _NSPLIT
    idx2d = x.reshape(n_rows // _CHUNK, _CHUNK)
    gathered = [
        _sc_gather(tok, idx2d, c, _NSPLIT) for c in range(_NSPLIT)
    ]
    blocks_per_split = rows_per_split // _SEQ
    out = None
    for c in range(_NSPLIT):
        out = _tc_pos_ln_chunk(
            gathered[c], pos, ln_w, ln_b, out, c * blocks_per_split, n_rows
        )
    return out.reshape(b, seq, _DIM)
